# Initial kernel scaffold; baseline (speedup 1.0000x reference)
import jax
import jax.numpy as jnp
from jax.experimental import pallas as pl


def kernel(x, edge_index, W, b):
    raise NotImplementedError("write your pallas kernel here")


# speedup vs baseline: 17.6277x; 17.6277x over previous
import functools

import jax
import jax.numpy as jnp
from jax import lax
from jax.experimental import pallas as pl
from jax.experimental.pallas import tpu as pltpu
from jax.experimental.pallas import tpu_sc as plsc

N = 10000
E = 320000
CH = 128

NC = 2
NS = 16
NW = NC * NS

CHUNK = 128
EPW_CHUNKS = -(-E // (NW * CHUNK))
EPW = EPW_CHUNKS * CHUNK
E_PAD = EPW * NW
N_PAD = 10112
RPT = N_PAD // NS
DUMMY = N

_mesh = plsc.VectorSubcoreMesh(core_axis_name="c", subcore_axis_name="s")
_sc_params = pltpu.CompilerParams(needs_layout_passes=False)


@functools.partial(
    pl.kernel,
    out_type=jax.ShapeDtypeStruct((NW, N_PAD), jnp.float32),
    mesh=_mesh,
    compiler_params=_sc_params,
    scratch_types=[
        pltpu.VMEM((EPW,), jnp.int32),
        pltpu.VMEM((N_PAD,), jnp.float32),
    ],
)
def _deg_kernel(dst_hbm, out_hbm, dstv, degv):
    wid = lax.axis_index("s") * NC + lax.axis_index("c")
    pltpu.sync_copy(dst_hbm.at[wid], dstv)

    zeros16 = jnp.zeros((16,), jnp.float32)
    ones16 = jnp.ones((16,), jnp.float32)

    def zero_body(i, _):
        degv[pl.ds(i * 16, 16)] = zeros16
        return 0

    lax.fori_loop(0, N_PAD // 16, zero_body, 0)

    def hist_body(i, _):
        idx = dstv[pl.ds(i * 16, 16)]
        plsc.addupdate_scatter(degv, [idx], ones16)
        return 0

    lax.fori_loop(0, EPW // 16, hist_body, 0)
    pltpu.sync_copy(degv, out_hbm.at[wid])


@functools.partial(
    pl.kernel,
    out_type=jax.ShapeDtypeStruct((NC, N_PAD, CH), jnp.float32),
    mesh=_mesh,
    compiler_params=_sc_params,
    scratch_types=[
        pltpu.VMEM((EPW_CHUNKS, CHUNK), jnp.int32),
        pltpu.VMEM((EPW_CHUNKS, CHUNK), jnp.int32),
        pltpu.VMEM((CHUNK, CH), jnp.float32),
        pltpu.VMEM_SHARED((N_PAD, CH), jnp.float32),
        pltpu.SemaphoreType.DMA,
    ],
)
def _scatter_kernel(y_hbm, src_hbm, dst_hbm, acc_hbm,
                    srcv, dstv, gbuf0, acc_sh, sem0):
    c = lax.axis_index("c")
    s = lax.axis_index("s")
    wid = s * NC + c
    pltpu.sync_copy(src_hbm.at[wid], srcv)
    pltpu.sync_copy(dst_hbm.at[wid], dstv)

    zeros16 = jnp.zeros((16,), jnp.float32)

    def zero_body(i, _):
        gbuf0[i // (CH // 16), pl.ds((i % (CH // 16)) * 16, 16)] = zeros16
        return 0

    lax.fori_loop(0, CHUNK * CH // 16, zero_body, 0)

    row0 = s * RPT
    pltpu.sync_copy(gbuf0, acc_sh.at[pl.ds(row0, CHUNK)])
    pltpu.sync_copy(gbuf0, acc_sh.at[pl.ds(row0 + CHUNK, CHUNK)])
    pltpu.sync_copy(gbuf0, acc_sh.at[pl.ds(row0 + 2 * CHUNK, CHUNK)])
    pltpu.sync_copy(gbuf0, acc_sh.at[pl.ds(row0 + 3 * CHUNK, CHUNK)])
    pltpu.sync_copy(gbuf0.at[pl.ds(0, RPT - 4 * CHUNK)],
                    acc_sh.at[pl.ds(row0 + 4 * CHUNK, RPT - 4 * CHUNK)])
    plsc.subcore_barrier()

    def body(j, _):
        pltpu.async_copy(y_hbm.at[srcv.at[j]], gbuf0, sem0).wait()
        pltpu.sync_copy(gbuf0, acc_sh.at[dstv.at[j]], add=True)
        return 0

    lax.fori_loop(0, EPW_CHUNKS, body, 0)
    plsc.subcore_barrier()

    pltpu.sync_copy(acc_sh.at[pl.ds(row0, RPT)],
                    acc_hbm.at[c, pl.ds(row0, RPT)])


def _linear_body(x_ref, w_ref, deg_ref, y_ref):
    deg = jnp.sum(deg_ref[...], axis=1) + 1.0
    dinv = lax.rsqrt(deg)
    xw = jnp.dot(x_ref[...], w_ref[...], preferred_element_type=jnp.float32)
    y_ref[...] = dinv[:, None] * xw


def _combine_body(acc_ref, y_ref, deg_ref, b_ref, o_ref):
    deg = jnp.sum(deg_ref[...], axis=1) + 1.0
    dinv = lax.rsqrt(deg)
    total = acc_ref[0] + acc_ref[1] + y_ref[...]
    o_ref[...] = dinv[:, None] * total + b_ref[...]


_ROWS_BLK = 1000


def kernel(x, edge_index, W, b):
    src = edge_index[0].astype(jnp.int32)
    dst = edge_index[1].astype(jnp.int32)
    pad = E_PAD - E
    src_p = jnp.concatenate([src, jnp.zeros((pad,), jnp.int32)])
    dst_p = jnp.concatenate([dst, jnp.full((pad,), DUMMY, jnp.int32)])

    deg_part = _deg_kernel(dst_p.reshape(NW, EPW)).T

    y = pl.pallas_call(
        _linear_body,
        grid=(N // _ROWS_BLK,),
        in_specs=[
            pl.BlockSpec((_ROWS_BLK, CH), lambda i: (i, 0)),
            pl.BlockSpec((CH, CH), lambda i: (0, 0)),
            pl.BlockSpec((_ROWS_BLK, NW), lambda i: (i, 0)),
        ],
        out_specs=pl.BlockSpec((_ROWS_BLK, CH), lambda i: (i, 0)),
        out_shape=jax.ShapeDtypeStruct((N, CH), jnp.float32),
    )(x, W, deg_part)

    acc = _scatter_kernel(y, src_p.reshape(NW, EPW_CHUNKS, CHUNK),
                          dst_p.reshape(NW, EPW_CHUNKS, CHUNK))

    out = pl.pallas_call(
        _combine_body,
        grid=(N // _ROWS_BLK,),
        in_specs=[
            pl.BlockSpec((NC, _ROWS_BLK, CH), lambda i: (0, i, 0)),
            pl.BlockSpec((_ROWS_BLK, CH), lambda i: (i, 0)),
            pl.BlockSpec((_ROWS_BLK, NW), lambda i: (i, 0)),
            pl.BlockSpec((1, CH), lambda i: (0, 0)),
        ],
        out_specs=pl.BlockSpec((_ROWS_BLK, CH), lambda i: (i, 0)),
        out_shape=jax.ShapeDtypeStruct((N, CH), jnp.float32),
    )(acc, y, deg_part, b.reshape(1, CH))

    return out


# speedup vs baseline: 17.7417x; 1.0065x over previous
import functools

import jax
import jax.numpy as jnp
from jax import lax
from jax.experimental import pallas as pl
from jax.experimental.pallas import tpu as pltpu
from jax.experimental.pallas import tpu_sc as plsc

N = 10000
E = 320000
CH = 128

NC = 2
NS = 16
NW = NC * NS

CHUNK = 128
EPW_CHUNKS = -(-E // (NW * CHUNK))
EPW = EPW_CHUNKS * CHUNK
E_PAD = EPW * NW
N_PAD = 10112
RPT = N_PAD // NS
DUMMY = N

_mesh = plsc.VectorSubcoreMesh(core_axis_name="c", subcore_axis_name="s")
_sc_params = pltpu.CompilerParams(needs_layout_passes=False)


@functools.partial(
    pl.kernel,
    out_type=jax.ShapeDtypeStruct((NW, N_PAD), jnp.float32),
    mesh=_mesh,
    compiler_params=_sc_params,
    scratch_types=[
        pltpu.VMEM((EPW,), jnp.int32),
        pltpu.VMEM((N_PAD,), jnp.float32),
    ],
)
def _deg_kernel(dst_hbm, out_hbm, dstv, degv):
    wid = lax.axis_index("s") * NC + lax.axis_index("c")
    pltpu.sync_copy(dst_hbm.at[wid], dstv)

    zeros16 = jnp.zeros((16,), jnp.float32)
    ones16 = jnp.ones((16,), jnp.float32)

    def zero_body(i, _):
        degv[pl.ds(i * 16, 16)] = zeros16
        return 0

    lax.fori_loop(0, N_PAD // 16, zero_body, 0)

    def hist_body(i, _):
        idx = dstv[pl.ds(i * 16, 16)]
        plsc.addupdate_scatter(degv, [idx], ones16)
        return 0

    lax.fori_loop(0, EPW // 16, hist_body, 0)
    pltpu.sync_copy(degv, out_hbm.at[wid])


@functools.partial(
    pl.kernel,
    out_type=jax.ShapeDtypeStruct((NC, N_PAD, CH), jnp.float32),
    mesh=_mesh,
    compiler_params=_sc_params,
    scratch_types=[
        pltpu.VMEM((EPW_CHUNKS, CHUNK), jnp.int32),
        pltpu.VMEM((EPW_CHUNKS, CHUNK), jnp.int32),
        pltpu.VMEM((CHUNK, CH), jnp.float32),
        pltpu.VMEM_SHARED((N_PAD, CH), jnp.float32),
        pltpu.SemaphoreType.DMA,
    ],
)
def _scatter_kernel(y_hbm, src_hbm, dst_hbm, acc_hbm,
                    srcv, dstv, gbuf0, acc_sh, sem0):
    c = lax.axis_index("c")
    s = lax.axis_index("s")
    wid = s * NC + c
    pltpu.sync_copy(src_hbm.at[wid], srcv)
    pltpu.sync_copy(dst_hbm.at[wid], dstv)

    zeros16 = jnp.zeros((16,), jnp.float32)

    def zero_body(i, _):
        gbuf0[i // (CH // 16), pl.ds((i % (CH // 16)) * 16, 16)] = zeros16
        return 0

    lax.fori_loop(0, CHUNK * CH // 16, zero_body, 0)

    row0 = s * RPT
    pltpu.sync_copy(gbuf0, acc_sh.at[pl.ds(row0, CHUNK)])
    pltpu.sync_copy(gbuf0, acc_sh.at[pl.ds(row0 + CHUNK, CHUNK)])
    pltpu.sync_copy(gbuf0, acc_sh.at[pl.ds(row0 + 2 * CHUNK, CHUNK)])
    pltpu.sync_copy(gbuf0, acc_sh.at[pl.ds(row0 + 3 * CHUNK, CHUNK)])
    pltpu.sync_copy(gbuf0.at[pl.ds(0, RPT - 4 * CHUNK)],
                    acc_sh.at[pl.ds(row0 + 4 * CHUNK, RPT - 4 * CHUNK)])
    plsc.subcore_barrier()

    def body(j, _):
        pltpu.async_copy(y_hbm.at[srcv.at[j]], gbuf0, sem0).wait()
        pltpu.sync_copy(gbuf0, acc_sh.at[dstv.at[j]], add=True)
        return 0

    lax.fori_loop(0, EPW_CHUNKS, body, 0)
    plsc.subcore_barrier()

    pltpu.sync_copy(acc_sh.at[pl.ds(row0, RPT)],
                    acc_hbm.at[c, pl.ds(row0, RPT)])


def _linear_body(x_ref, w_ref, deg_ref, y_ref):
    deg = jnp.sum(deg_ref[...], axis=1) + 1.0
    dinv = lax.rsqrt(deg)
    xw = jnp.dot(x_ref[...], w_ref[...], preferred_element_type=jnp.float32)
    y_ref[...] = dinv[:, None] * xw


def _combine_body(acc_ref, y_ref, deg_ref, b_ref, o_ref):
    deg = jnp.sum(deg_ref[...], axis=1) + 1.0
    dinv = lax.rsqrt(deg)
    total = acc_ref[0] + acc_ref[1] + y_ref[...]
    o_ref[...] = dinv[:, None] * total + b_ref[...]


_ROWS_BLK = 1000


def kernel(x, edge_index, W, b):
    src = edge_index[0].astype(jnp.int32)
    dst = edge_index[1].astype(jnp.int32)
    pad = E_PAD - E
    src_p = jnp.concatenate([src, jnp.zeros((pad,), jnp.int32)])
    dst_pad = DUMMY + (jnp.arange(pad, dtype=jnp.int32) % (N_PAD - N))
    dst_p = jnp.concatenate([dst, dst_pad])

    deg_part = _deg_kernel(dst_p.reshape(NW, EPW)).T

    y = pl.pallas_call(
        _linear_body,
        grid=(N // _ROWS_BLK,),
        in_specs=[
            pl.BlockSpec((_ROWS_BLK, CH), lambda i: (i, 0)),
            pl.BlockSpec((CH, CH), lambda i: (0, 0)),
            pl.BlockSpec((_ROWS_BLK, NW), lambda i: (i, 0)),
        ],
        out_specs=pl.BlockSpec((_ROWS_BLK, CH), lambda i: (i, 0)),
        out_shape=jax.ShapeDtypeStruct((N, CH), jnp.float32),
    )(x, W, deg_part)

    acc = _scatter_kernel(y, src_p.reshape(NW, EPW_CHUNKS, CHUNK),
                          dst_p.reshape(NW, EPW_CHUNKS, CHUNK))

    out = pl.pallas_call(
        _combine_body,
        grid=(N // _ROWS_BLK,),
        in_specs=[
            pl.BlockSpec((NC, _ROWS_BLK, CH), lambda i: (0, i, 0)),
            pl.BlockSpec((_ROWS_BLK, CH), lambda i: (i, 0)),
            pl.BlockSpec((_ROWS_BLK, NW), lambda i: (i, 0)),
            pl.BlockSpec((1, CH), lambda i: (0, 0)),
        ],
        out_specs=pl.BlockSpec((_ROWS_BLK, CH), lambda i: (i, 0)),
        out_shape=jax.ShapeDtypeStruct((N, CH), jnp.float32),
    )(acc, y, deg_part, b.reshape(1, CH))

    return out


# speedup vs baseline: 19.7678x; 1.1142x over previous
import functools

import jax
import jax.numpy as jnp
from jax import lax
from jax.experimental import pallas as pl
from jax.experimental.pallas import tpu as pltpu
from jax.experimental.pallas import tpu_sc as plsc

N = 10000
E = 320000
CH = 128

NC = 2
NS = 16
NW = NC * NS

CHUNK = 128
EPW_CHUNKS = -(-E // (NW * CHUNK))
EPW = EPW_CHUNKS * CHUNK
E_PAD = EPW * NW
N_PAD = 10112
RPT = N_PAD // NS
DUMMY = N

_mesh = plsc.VectorSubcoreMesh(core_axis_name="c", subcore_axis_name="s")
_sc_params = pltpu.CompilerParams(needs_layout_passes=False)


@functools.partial(
    pl.kernel,
    out_type=jax.ShapeDtypeStruct((NW, N_PAD), jnp.float32),
    mesh=_mesh,
    compiler_params=_sc_params,
    scratch_types=[
        pltpu.VMEM((EPW,), jnp.int32),
        pltpu.VMEM((N_PAD,), jnp.float32),
    ],
)
def _deg_kernel(dst_hbm, out_hbm, dstv, degv):
    wid = lax.axis_index("s") * NC + lax.axis_index("c")
    pltpu.sync_copy(dst_hbm.at[wid], dstv)

    zeros16 = jnp.zeros((16,), jnp.float32)
    ones16 = jnp.ones((16,), jnp.float32)

    def zero_body(i, _):
        degv[pl.ds(i * 16, 16)] = zeros16
        return 0

    lax.fori_loop(0, N_PAD // 16, zero_body, 0)

    def hist_body(i, _):
        idx = dstv[pl.ds(i * 16, 16)]
        plsc.addupdate_scatter(degv, [idx], ones16)
        return 0

    lax.fori_loop(0, EPW // 16, hist_body, 0)
    pltpu.sync_copy(degv, out_hbm.at[wid])


@functools.partial(
    pl.kernel,
    out_type=jax.ShapeDtypeStruct((NC, N_PAD, CH), jnp.float32),
    mesh=_mesh,
    compiler_params=_sc_params,
    scratch_types=[
        pltpu.VMEM((3, CHUNK), jnp.int32),
        pltpu.VMEM((3, CHUNK), jnp.int32),
        pltpu.VMEM((2, CHUNK, CH), jnp.float32),
        pltpu.VMEM_SHARED((N_PAD, CH), jnp.float32),
        pltpu.SemaphoreType.DMA,
        pltpu.SemaphoreType.DMA,
    ],
)
def _scatter_kernel(y_hbm, src_hbm, dst_hbm, acc_hbm,
                    sidx, didx, gbuf, acc_sh, semI, semG):
    c = lax.axis_index("c")
    s = lax.axis_index("s")
    wid = s * NC + c

    pltpu.sync_copy(src_hbm.at[wid, 0], sidx.at[0])
    pltpu.sync_copy(dst_hbm.at[wid, 0], didx.at[0])
    pltpu.async_copy(y_hbm.at[sidx.at[0]], gbuf.at[0], semG)
    pltpu.async_copy(src_hbm.at[wid, 1], sidx.at[1], semI)
    pltpu.async_copy(dst_hbm.at[wid, 1], didx.at[1], semI)

    zeros16 = jnp.zeros((16,), jnp.float32)

    def zero_body(i, _):
        gbuf[1, i // (CH // 16), pl.ds((i % (CH // 16)) * 16, 16)] = zeros16
        return 0

    lax.fori_loop(0, CHUNK * CH // 16, zero_body, 0)

    row0 = s * RPT
    pltpu.sync_copy(gbuf.at[1], acc_sh.at[pl.ds(row0, CHUNK)])
    pltpu.sync_copy(gbuf.at[1], acc_sh.at[pl.ds(row0 + CHUNK, CHUNK)])
    pltpu.sync_copy(gbuf.at[1], acc_sh.at[pl.ds(row0 + 2 * CHUNK, CHUNK)])
    pltpu.sync_copy(gbuf.at[1], acc_sh.at[pl.ds(row0 + 3 * CHUNK, CHUNK)])
    pltpu.sync_copy(gbuf.at[1, pl.ds(0, RPT - 4 * CHUNK)],
                    acc_sh.at[pl.ds(row0 + 4 * CHUNK, RPT - 4 * CHUNK)])
    plsc.subcore_barrier()

    def body(j, _):
        cur = j % 2
        nxt = 1 - cur
        cur3 = j % 3
        pltpu.make_async_copy(y_hbm.at[sidx.at[cur3]], gbuf.at[cur],
                              semG).wait()

        @pl.when(j + 1 < EPW_CHUNKS)
        def _():
            nxt3 = (j + 1) % 3
            pltpu.make_async_copy(src_hbm.at[wid, j + 1], sidx.at[nxt3],
                                  semI).wait()
            pltpu.make_async_copy(dst_hbm.at[wid, j + 1], didx.at[nxt3],
                                  semI).wait()
            pltpu.async_copy(y_hbm.at[sidx.at[nxt3]], gbuf.at[nxt], semG)

        pltpu.sync_copy(gbuf.at[cur], acc_sh.at[didx.at[cur3]], add=True)

        @pl.when(j + 2 < EPW_CHUNKS)
        def _():
            n2 = (j + 2) % 3
            pltpu.async_copy(src_hbm.at[wid, j + 2], sidx.at[n2], semI)
            pltpu.async_copy(dst_hbm.at[wid, j + 2], didx.at[n2], semI)

        return 0

    lax.fori_loop(0, EPW_CHUNKS, body, 0)
    plsc.subcore_barrier()

    pltpu.sync_copy(acc_sh.at[pl.ds(row0, RPT)],
                    acc_hbm.at[c, pl.ds(row0, RPT)])


def _linear_body(x_ref, w_ref, deg_ref, y_ref):
    deg = jnp.sum(deg_ref[...], axis=1) + 1.0
    dinv = lax.rsqrt(deg)
    xw = jnp.dot(x_ref[...], w_ref[...], preferred_element_type=jnp.float32)
    y_ref[...] = dinv[:, None] * xw


def _combine_body(acc_ref, y_ref, deg_ref, b_ref, o_ref):
    deg = jnp.sum(deg_ref[...], axis=1) + 1.0
    dinv = lax.rsqrt(deg)
    total = acc_ref[0] + acc_ref[1] + y_ref[...]
    o_ref[...] = dinv[:, None] * total + b_ref[...]


_ROWS_BLK = 1000


def kernel(x, edge_index, W, b):
    src = edge_index[0].astype(jnp.int32)
    dst = edge_index[1].astype(jnp.int32)
    pad = E_PAD - E
    src_p = jnp.concatenate([src, jnp.zeros((pad,), jnp.int32)])
    dst_pad = DUMMY + (jnp.arange(pad, dtype=jnp.int32) % (N_PAD - N))
    dst_p = jnp.concatenate([dst, dst_pad])

    deg_part = _deg_kernel(dst_p.reshape(NW, EPW)).T

    y = pl.pallas_call(
        _linear_body,
        grid=(N // _ROWS_BLK,),
        in_specs=[
            pl.BlockSpec((_ROWS_BLK, CH), lambda i: (i, 0)),
            pl.BlockSpec((CH, CH), lambda i: (0, 0)),
            pl.BlockSpec((_ROWS_BLK, NW), lambda i: (i, 0)),
        ],
        out_specs=pl.BlockSpec((_ROWS_BLK, CH), lambda i: (i, 0)),
        out_shape=jax.ShapeDtypeStruct((N, CH), jnp.float32),
    )(x, W, deg_part)

    acc = _scatter_kernel(y, src_p.reshape(NW, EPW_CHUNKS, CHUNK),
                          dst_p.reshape(NW, EPW_CHUNKS, CHUNK))

    out = pl.pallas_call(
        _combine_body,
        grid=(N // _ROWS_BLK,),
        in_specs=[
            pl.BlockSpec((NC, _ROWS_BLK, CH), lambda i: (0, i, 0)),
            pl.BlockSpec((_ROWS_BLK, CH), lambda i: (i, 0)),
            pl.BlockSpec((_ROWS_BLK, NW), lambda i: (i, 0)),
            pl.BlockSpec((1, CH), lambda i: (0, 0)),
        ],
        out_specs=pl.BlockSpec((_ROWS_BLK, CH), lambda i: (i, 0)),
        out_shape=jax.ShapeDtypeStruct((N, CH), jnp.float32),
    )(acc, y, deg_part, b.reshape(1, CH))

    return out


# speedup vs baseline: 22.0142x; 1.1136x over previous
import functools

import jax
import jax.numpy as jnp
from jax import lax
from jax.experimental import pallas as pl
from jax.experimental.pallas import tpu as pltpu
from jax.experimental.pallas import tpu_sc as plsc

N = 10000
E = 320000
CH = 128

NC = 2
NS = 16
NW = NC * NS

CHUNK = 128
EPW_CHUNKS = -(-E // (NW * CHUNK))
EPW = EPW_CHUNKS * CHUNK
E_PAD = EPW * NW
N_PAD = 10112
RPT = N_PAD // NS
DUMMY = N

_mesh = plsc.VectorSubcoreMesh(core_axis_name="c", subcore_axis_name="s")
_sc_params = pltpu.CompilerParams(needs_layout_passes=False)


@functools.partial(
    pl.kernel,
    out_type=jax.ShapeDtypeStruct((NW, N_PAD), jnp.float32),
    mesh=_mesh,
    compiler_params=_sc_params,
    scratch_types=[
        pltpu.VMEM((EPW,), jnp.int32),
        pltpu.VMEM((N_PAD,), jnp.float32),
    ],
)
def _deg_kernel(dst_hbm, out_hbm, dstv, degv):
    wid = lax.axis_index("s") * NC + lax.axis_index("c")
    pltpu.sync_copy(dst_hbm.at[wid], dstv)

    zeros16 = jnp.zeros((16,), jnp.float32)
    ones16 = jnp.ones((16,), jnp.float32)

    def zero_body(i, _):
        degv[pl.ds(i * 16, 16)] = zeros16
        return 0

    lax.fori_loop(0, N_PAD // 16, zero_body, 0)

    def hist_body(i, _):
        idx = dstv[pl.ds(i * 16, 16)]
        plsc.addupdate_scatter(degv, [idx], ones16)
        return 0

    lax.fori_loop(0, EPW // 16, hist_body, 0)
    pltpu.sync_copy(degv, out_hbm.at[wid])


@functools.partial(
    pl.kernel,
    out_type=jax.ShapeDtypeStruct((NC, N_PAD, CH), jnp.float32),
    mesh=_mesh,
    compiler_params=_sc_params,
    scratch_types=[
        pltpu.VMEM((3, CHUNK), jnp.int32),
        pltpu.VMEM((3, CHUNK), jnp.int32),
        pltpu.VMEM((2, CHUNK, CH), jnp.float32),
        pltpu.VMEM_SHARED((N_PAD, CH), jnp.float32),
        pltpu.SemaphoreType.DMA,
        pltpu.SemaphoreType.DMA,
    ],
)
def _scatter_kernel(ya_hbm, yb_hbm, src_hbm, dst_hbm, acc_hbm,
                    sidx, didx, gbuf, acc_sh, semI, semG):
    c = lax.axis_index("c")
    s = lax.axis_index("s")
    wid = s * NC + c

    pltpu.sync_copy(src_hbm.at[wid, 0], sidx.at[0])
    pltpu.sync_copy(dst_hbm.at[wid, 0], didx.at[0])

    @pl.when(c == 0)
    def _():
        pltpu.async_copy(ya_hbm.at[sidx.at[0]], gbuf.at[0], semG)

    @pl.when(c == 1)
    def _():
        pltpu.async_copy(yb_hbm.at[sidx.at[0]], gbuf.at[0], semG)

    pltpu.async_copy(src_hbm.at[wid, 1], sidx.at[1], semI)
    pltpu.async_copy(dst_hbm.at[wid, 1], didx.at[1], semI)

    zeros16 = jnp.zeros((16,), jnp.float32)

    def zero_body(i, _):
        gbuf[1, i // (CH // 16), pl.ds((i % (CH // 16)) * 16, 16)] = zeros16
        return 0

    lax.fori_loop(0, CHUNK * CH // 16, zero_body, 0)

    row0 = s * RPT
    pltpu.sync_copy(gbuf.at[1], acc_sh.at[pl.ds(row0, CHUNK)])
    pltpu.sync_copy(gbuf.at[1], acc_sh.at[pl.ds(row0 + CHUNK, CHUNK)])
    pltpu.sync_copy(gbuf.at[1], acc_sh.at[pl.ds(row0 + 2 * CHUNK, CHUNK)])
    pltpu.sync_copy(gbuf.at[1], acc_sh.at[pl.ds(row0 + 3 * CHUNK, CHUNK)])
    pltpu.sync_copy(gbuf.at[1, pl.ds(0, RPT - 4 * CHUNK)],
                    acc_sh.at[pl.ds(row0 + 4 * CHUNK, RPT - 4 * CHUNK)])
    plsc.subcore_barrier()

    def _make_body(y_hbm):
        def body(j, _):
            cur = j % 2
            nxt = 1 - cur
            cur3 = j % 3
            pltpu.make_async_copy(y_hbm.at[sidx.at[cur3]], gbuf.at[cur],
                                  semG).wait()

            @pl.when(j + 1 < EPW_CHUNKS)
            def _():
                nxt3 = (j + 1) % 3
                pltpu.make_async_copy(src_hbm.at[wid, j + 1], sidx.at[nxt3],
                                      semI).wait()
                pltpu.make_async_copy(dst_hbm.at[wid, j + 1], didx.at[nxt3],
                                      semI).wait()
                pltpu.async_copy(y_hbm.at[sidx.at[nxt3]], gbuf.at[nxt], semG)

            pltpu.sync_copy(gbuf.at[cur], acc_sh.at[didx.at[cur3]], add=True)

            @pl.when(j + 2 < EPW_CHUNKS)
            def _():
                n2 = (j + 2) % 3
                pltpu.async_copy(src_hbm.at[wid, j + 2], sidx.at[n2], semI)
                pltpu.async_copy(dst_hbm.at[wid, j + 2], didx.at[n2], semI)

            return 0

        return body

    @pl.when(c == 0)
    def _():
        lax.fori_loop(0, EPW_CHUNKS, _make_body(ya_hbm), 0)

    @pl.when(c == 1)
    def _():
        lax.fori_loop(0, EPW_CHUNKS, _make_body(yb_hbm), 0)

    plsc.subcore_barrier()

    pltpu.sync_copy(acc_sh.at[pl.ds(row0, RPT)],
                    acc_hbm.at[c, pl.ds(row0, RPT)])


def _linear_body(x_ref, w_ref, deg_ref, y_ref, yb_ref):
    deg = jnp.sum(deg_ref[...], axis=1) + 1.0
    dinv = lax.rsqrt(deg)
    xw = jnp.dot(x_ref[...], w_ref[...], preferred_element_type=jnp.float32)
    y = dinv[:, None] * xw
    y_ref[...] = y
    yb_ref[...] = y


def _combine_body(acc_ref, y_ref, deg_ref, b_ref, o_ref):
    deg = jnp.sum(deg_ref[...], axis=1) + 1.0
    dinv = lax.rsqrt(deg)
    total = acc_ref[0] + acc_ref[1] + y_ref[...]
    o_ref[...] = dinv[:, None] * total + b_ref[...]


_ROWS_BLK = 1000


def kernel(x, edge_index, W, b):
    src = edge_index[0].astype(jnp.int32)
    dst = edge_index[1].astype(jnp.int32)
    pad = E_PAD - E
    src_p = jnp.concatenate([src, jnp.zeros((pad,), jnp.int32)])
    dst_pad = DUMMY + (jnp.arange(pad, dtype=jnp.int32) % (N_PAD - N))
    dst_p = jnp.concatenate([dst, dst_pad])

    deg_part = _deg_kernel(dst_p.reshape(NW, EPW)).T

    y, yb = pl.pallas_call(
        _linear_body,
        grid=(N // _ROWS_BLK,),
        in_specs=[
            pl.BlockSpec((_ROWS_BLK, CH), lambda i: (i, 0)),
            pl.BlockSpec((CH, CH), lambda i: (0, 0)),
            pl.BlockSpec((_ROWS_BLK, NW), lambda i: (i, 0)),
        ],
        out_specs=[
            pl.BlockSpec((_ROWS_BLK, CH), lambda i: (i, 0)),
            pl.BlockSpec((_ROWS_BLK, CH), lambda i: (i, 0)),
        ],
        out_shape=[
            jax.ShapeDtypeStruct((N, CH), jnp.float32),
            jax.ShapeDtypeStruct((N, CH), jnp.float32),
        ],
    )(x, W, deg_part)

    acc = _scatter_kernel(y, yb, src_p.reshape(NW, EPW_CHUNKS, CHUNK),
                          dst_p.reshape(NW, EPW_CHUNKS, CHUNK))

    out = pl.pallas_call(
        _combine_body,
        grid=(N // _ROWS_BLK,),
        in_specs=[
            pl.BlockSpec((NC, _ROWS_BLK, CH), lambda i: (0, i, 0)),
            pl.BlockSpec((_ROWS_BLK, CH), lambda i: (i, 0)),
            pl.BlockSpec((_ROWS_BLK, NW), lambda i: (i, 0)),
            pl.BlockSpec((1, CH), lambda i: (0, 0)),
        ],
        out_specs=pl.BlockSpec((_ROWS_BLK, CH), lambda i: (i, 0)),
        out_shape=jax.ShapeDtypeStruct((N, CH), jnp.float32),
    )(acc, y, deg_part, b.reshape(1, CH))

    return out
